# baseline (speedup 1.0000x reference)
import jax
import jax.numpy as jnp
from jax.experimental import pallas as pl
from jax.experimental.pallas import tpu as pltpu

N_GRAPHS = 512


def _mlp_body(pooled_ref, w1, b1, w2, b2, w3, b3, w4, b4, out_ref):
    r = pooled_ref[...]
    r = jnp.maximum(r @ w1[...] + b1[...], 0.0)
    r = jnp.maximum(r @ w2[...] + b2[...], 0.0)
    r = jnp.maximum(r @ w3[...] + b3[...], 0.0)
    out_ref[...] = r @ w4[...] + b4[...]


def kernel(x, edge_index, batch, W1, b1, W2, b2, Wr1, br1, Wr2, br2, Wr3, br3, Wr4, br4):
    n = x.shape[0]
    loop = jnp.arange(n, dtype=edge_index.dtype)
    src = jnp.concatenate([edge_index[0], loop])
    dst = jnp.concatenate([edge_index[1], loop])
    deg = jax.ops.segment_sum(jnp.ones_like(dst, dtype=jnp.float32), dst, num_segments=n)
    dis = jnp.where(deg > 0, deg ** -0.5, 0.0)
    norm = dis[src] * dis[dst]

    def conv(x, W, b):
        h = x @ W
        msgs = h[src] * norm[:, None]
        return jax.ops.segment_sum(msgs, dst, num_segments=n) + b

    h = jax.nn.relu(conv(x, W1, b1))
    h = jax.nn.relu(conv(h, W2, b2))
    sums = jax.ops.segment_sum(h, batch, num_segments=N_GRAPHS)
    counts = jax.ops.segment_sum(jnp.ones((n,), jnp.float32), batch, num_segments=N_GRAPHS)
    pooled = sums / jnp.maximum(counts, 1.0)[:, None]

    out = pl.pallas_call(
        _mlp_body,
        out_shape=jax.ShapeDtypeStruct((N_GRAPHS, 1), jnp.float32),
    )(pooled, Wr1, br1, Wr2, br2, Wr3, br3, Wr4, br4)
    return out


# speedup vs baseline: 50.9333x; 50.9333x over previous
import functools

import jax
import jax.numpy as jnp
from jax import lax
from jax.experimental import pallas as pl
from jax.experimental.pallas import tpu as pltpu
from jax.experimental.pallas import tpu_sc as plsc

NC = 2
NS = 16
NW = NC * NS
L = 16

N = 50000
E = 1_600_000
NG = 512

C = 128
CPT = 392
KB = 56
EPAD = NW * CPT * C
NP = 51200
RPS = NP // NS

BT = 1000


def _sc_mesh():
    return plsc.VectorSubcoreMesh(
        core_axis_name="c", subcore_axis_name="s",
        num_cores=NC, num_subcores=NS)


def _sc_degree(dstp):
    @functools.partial(
        pl.kernel,
        out_type=jax.ShapeDtypeStruct((NC, NP), jnp.float32),
        mesh=_sc_mesh(),
        compiler_params=pltpu.CompilerParams(use_tc_tiling_on_sc=False),
        scratch_types=[
            pltpu.VMEM_SHARED((NP,), jnp.float32),
            pltpu.VMEM((CPT, C), jnp.int32),
            pltpu.VMEM((C,), jnp.float32),
            pltpu.VMEM((C,), jnp.float32),
        ],
    )
    def k(dst_hbm, out_hbm, acc, dst_i, ones, zeros):
        cid = lax.axis_index("c")
        sid = lax.axis_index("s")
        wid = cid * NS + sid

        def fill(i, _):
            ones[pl.ds(i * L, L)] = jnp.full((L,), 1.0, jnp.float32)
            zeros[pl.ds(i * L, L)] = jnp.zeros((L,), jnp.float32)
            return 0
        lax.fori_loop(0, C // L, fill, 0)

        def zero(i, _):
            pltpu.sync_copy(zeros, acc.at[pl.ds(sid * RPS + i * C, C)])
            return 0
        lax.fori_loop(0, RPS // C, zero, 0)
        plsc.subcore_barrier()

        pltpu.sync_copy(dst_hbm.at[wid], dst_i)

        def body(b, _):
            pltpu.sync_copy(ones, acc.at[dst_i.at[b]], add=True)
            return 0
        lax.fori_loop(0, CPT, body, 0)
        plsc.subcore_barrier()

        pltpu.sync_copy(acc.at[pl.ds(sid * RPS, RPS)],
                        out_hbm.at[cid, pl.ds(sid * RPS, RPS)])

    return k(dstp)


def _sc_scatter(g, srcp, dstp, d):
    @functools.partial(
        pl.kernel,
        out_type=jax.ShapeDtypeStruct((NC, NP, d), jnp.float32),
        mesh=_sc_mesh(),
        compiler_params=pltpu.CompilerParams(use_tc_tiling_on_sc=False),
        scratch_types=[
            pltpu.VMEM_SHARED((NP, d), jnp.float32),
            pltpu.VMEM((KB, C), jnp.int32),
            pltpu.VMEM((KB, C), jnp.int32),
            pltpu.VMEM((C, d), jnp.float32),
            pltpu.VMEM((C, d), jnp.float32),
            pltpu.SemaphoreType.DMA,
            pltpu.SemaphoreType.DMA,
        ],
    )
    def k(g_hbm, src_hbm, dst_hbm, out_hbm,
          acc, src_i, dst_i, rows0, rows1, sem0, sem1):
        cid = lax.axis_index("c")
        sid = lax.axis_index("s")
        wid = cid * NS + sid

        def fill(i, _):
            rows0[i // (d // L), pl.ds((i % (d // L)) * L, L)] = (
                jnp.zeros((L,), jnp.float32))
            return 0
        lax.fori_loop(0, C * (d // L), fill, 0)

        def zero(i, _):
            pltpu.sync_copy(rows0, acc.at[pl.ds(sid * RPS + i * C, C)])
            return 0
        lax.fori_loop(0, RPS // C, zero, 0)
        plsc.subcore_barrier()

        def block(blk, _):
            pltpu.sync_copy(src_hbm.at[wid, pl.ds(blk * KB, KB)], src_i)
            pltpu.sync_copy(dst_hbm.at[wid, pl.ds(blk * KB, KB)], dst_i)
            pltpu.async_copy(g_hbm.at[src_i.at[0]], rows0, sem0)

            def pair(p, _):
                b0 = 2 * p
                pltpu.async_copy(g_hbm.at[src_i.at[b0 + 1]], rows1, sem1)
                pltpu.make_async_copy(g_hbm.at[pl.ds(0, C)], rows0, sem0).wait()
                pltpu.sync_copy(rows0, acc.at[dst_i.at[b0]], add=True)

                @pl.when(b0 + 2 < KB)
                def _():
                    pltpu.async_copy(g_hbm.at[src_i.at[b0 + 2]], rows0, sem0)

                pltpu.make_async_copy(g_hbm.at[pl.ds(0, C)], rows1, sem1).wait()
                pltpu.sync_copy(rows1, acc.at[dst_i.at[b0 + 1]], add=True)
                return 0
            lax.fori_loop(0, KB // 2, pair, 0)
            return 0
        lax.fori_loop(0, CPT // KB, block, 0)
        plsc.subcore_barrier()

        pltpu.sync_copy(acc.at[pl.ds(sid * RPS, RPS)],
                        out_hbm.at[cid, pl.ds(sid * RPS, RPS)])

    return k(g, srcp, dstp)


def _tc_prep(x, W1, deg0, deg1):
    def body(x_ref, w_ref, d0_ref, d1_ref, g_ref, dis_ref):
        deg = d0_ref[...] + d1_ref[...] + 1.0
        dis = lax.rsqrt(deg)
        g_ref[...] = jnp.dot(x_ref[...], w_ref[...],
                             preferred_element_type=jnp.float32) * dis
        dis_ref[...] = dis

    return pl.pallas_call(
        body,
        grid=(N // BT,),
        in_specs=[
            pl.BlockSpec((BT, 47), lambda i: (i, 0)),
            pl.BlockSpec((47, 16), lambda i: (0, 0)),
            pl.BlockSpec((BT, 1), lambda i: (i, 0)),
            pl.BlockSpec((BT, 1), lambda i: (i, 0)),
        ],
        out_specs=[
            pl.BlockSpec((BT, 16), lambda i: (i, 0)),
            pl.BlockSpec((BT, 1), lambda i: (i, 0)),
        ],
        out_shape=[
            jax.ShapeDtypeStruct((N, 16), jnp.float32),
            jax.ShapeDtypeStruct((N, 1), jnp.float32),
        ],
    )(x, W1, deg0, deg1)


def _tc_mid(s1p, g1, dis, b1, W2):
    def body(s_ref, g_ref, dis_ref, b_ref, w_ref, out_ref):
        s = s_ref[0] + s_ref[1] + g_ref[...]
        h = jnp.maximum(dis_ref[...] * s + b_ref[...], 0.0)
        out_ref[...] = jnp.dot(h, w_ref[...],
                               preferred_element_type=jnp.float32) * dis_ref[...]

    return pl.pallas_call(
        body,
        grid=(N // BT,),
        in_specs=[
            pl.BlockSpec((2, BT, 16), lambda i: (0, i, 0)),
            pl.BlockSpec((BT, 16), lambda i: (i, 0)),
            pl.BlockSpec((BT, 1), lambda i: (i, 0)),
            pl.BlockSpec((1, 16), lambda i: (0, 0)),
            pl.BlockSpec((16, 32), lambda i: (0, 0)),
        ],
        out_specs=pl.BlockSpec((BT, 32), lambda i: (i, 0)),
        out_shape=jax.ShapeDtypeStruct((N, 32), jnp.float32),
    )(s1p, g1, dis, b1.reshape(1, 16), W2)


def _tc_final(s2p, g2, dis, b2, batch_col,
              Wr1, br1, Wr2, br2, Wr3, br3, Wr4, br4):
    nb = N // BT

    def body(s_ref, g_ref, dis_ref, b_ref, bat_ref,
             w1, c1, w2, c2, w3, c3, w4, c4, out_ref, acc, cnt):
        i = pl.program_id(0)
        s = s_ref[0] + s_ref[1] + g_ref[...]
        h = jnp.maximum(dis_ref[...] * s + b_ref[...], 0.0)
        ids = lax.broadcasted_iota(jnp.int32, (1, NG), 1)
        a = (bat_ref[...] == ids).astype(jnp.float32)
        dn = (((0,), (0,)), ((), ()))
        contrib = lax.dot_general(a, h, dn,
                                  preferred_element_type=jnp.float32)
        ccontrib = lax.dot_general(a, jnp.ones((BT, 1), jnp.float32), dn,
                                   preferred_element_type=jnp.float32)

        @pl.when(i == 0)
        def _():
            acc[...] = contrib
            cnt[...] = ccontrib

        @pl.when(i > 0)
        def _():
            acc[...] += contrib
            cnt[...] += ccontrib

        @pl.when(i == nb - 1)
        def _():
            pooled = acc[...] / jnp.maximum(cnt[...], 1.0)
            r = jnp.maximum(jnp.dot(pooled, w1[...],
                                    preferred_element_type=jnp.float32) + c1[...], 0.0)
            r = jnp.maximum(jnp.dot(r, w2[...],
                                    preferred_element_type=jnp.float32) + c2[...], 0.0)
            r = jnp.maximum(jnp.dot(r, w3[...],
                                    preferred_element_type=jnp.float32) + c3[...], 0.0)
            out_ref[...] = jnp.dot(r, w4[...],
                                   preferred_element_type=jnp.float32) + c4[...]

    return pl.pallas_call(
        body,
        grid=(nb,),
        in_specs=[
            pl.BlockSpec((2, BT, 32), lambda i: (0, i, 0)),
            pl.BlockSpec((BT, 32), lambda i: (i, 0)),
            pl.BlockSpec((BT, 1), lambda i: (i, 0)),
            pl.BlockSpec((1, 32), lambda i: (0, 0)),
            pl.BlockSpec((BT, 1), lambda i: (i, 0)),
            pl.BlockSpec((32, 32), lambda i: (0, 0)),
            pl.BlockSpec((1, 32), lambda i: (0, 0)),
            pl.BlockSpec((32, 16), lambda i: (0, 0)),
            pl.BlockSpec((1, 16), lambda i: (0, 0)),
            pl.BlockSpec((16, 8), lambda i: (0, 0)),
            pl.BlockSpec((1, 8), lambda i: (0, 0)),
            pl.BlockSpec((8, 1), lambda i: (0, 0)),
            pl.BlockSpec((1, 1), lambda i: (0, 0)),
        ],
        out_specs=pl.BlockSpec((NG, 1), lambda i: (0, 0)),
        out_shape=jax.ShapeDtypeStruct((NG, 1), jnp.float32),
        scratch_shapes=[
            pltpu.VMEM((NG, 32), jnp.float32),
            pltpu.VMEM((NG, 1), jnp.float32),
        ],
    )(s2p, g2, dis, b2.reshape(1, 32), batch_col,
      Wr1, br1.reshape(1, 32), Wr2, br2.reshape(1, 16),
      Wr3, br3.reshape(1, 8), Wr4, br4.reshape(1, 1))


def kernel(x, edge_index, batch, W1, b1, W2, b2,
           Wr1, br1, Wr2, br2, Wr3, br3, Wr4, br4):
    src = edge_index[0].astype(jnp.int32)
    dst = edge_index[1].astype(jnp.int32)
    batch_col = batch.astype(jnp.int32).reshape(N, 1)

    npad = EPAD - E
    src_p = jnp.concatenate(
        [src, jnp.zeros((npad,), jnp.int32)]).reshape(NW, CPT, C)
    dst_p = jnp.concatenate(
        [dst, N + (jnp.arange(npad, dtype=jnp.int32) % (NP - N))]
    ).reshape(NW, CPT, C)

    degp = _sc_degree(dst_p)
    deg0 = degp[0].reshape(NP, 1)
    deg1 = degp[1].reshape(NP, 1)
    g1, dis = _tc_prep(x, W1, deg0, deg1)
    s1p = _sc_scatter(g1, src_p, dst_p, 16)
    g2 = _tc_mid(s1p, g1, dis, b1, W2)
    s2p = _sc_scatter(g2, src_p, dst_p, 32)
    return _tc_final(s2p, g2, dis, b2, batch_col,
                     Wr1, br1, Wr2, br2, Wr3, br3, Wr4, br4)


# speedup vs baseline: 57.7553x; 1.1339x over previous
import functools

import jax
import jax.numpy as jnp
from jax import lax
from jax.experimental import pallas as pl
from jax.experimental.pallas import tpu as pltpu
from jax.experimental.pallas import tpu_sc as plsc

NC = 2
NS = 16
NW = NC * NS
L = 16

N = 50000
E = 1_600_000
NG = 512

C = 128
CPT = 392
KB = 28
EPAD = NW * CPT * C
NP = 51200
RPS = NP // NS

BT = 1000


def _sc_mesh():
    return plsc.VectorSubcoreMesh(
        core_axis_name="c", subcore_axis_name="s",
        num_cores=NC, num_subcores=NS)


def _sc_degree(dstp):
    @functools.partial(
        pl.kernel,
        out_type=jax.ShapeDtypeStruct((NC, NP), jnp.float32),
        mesh=_sc_mesh(),
        compiler_params=pltpu.CompilerParams(use_tc_tiling_on_sc=False),
        scratch_types=[
            pltpu.VMEM_SHARED((NP,), jnp.float32),
            pltpu.VMEM((CPT, C), jnp.int32),
            pltpu.VMEM((C,), jnp.float32),
            pltpu.VMEM((C,), jnp.float32),
            pltpu.SemaphoreType.DMA,
        ],
    )
    def k(dst_hbm, out_hbm, acc, dst_i, ones, zeros, dsem):
        cid = lax.axis_index("c")
        sid = lax.axis_index("s")
        wid = cid * NS + sid

        def fill(i, _):
            ones[pl.ds(i * L, L)] = jnp.full((L,), 1.0, jnp.float32)
            zeros[pl.ds(i * L, L)] = jnp.zeros((L,), jnp.float32)
            return 0
        lax.fori_loop(0, C // L, fill, 0)

        def zero(i, _):
            pltpu.sync_copy(zeros, acc.at[pl.ds(sid * RPS + i * C, C)])
            return 0
        lax.fori_loop(0, RPS // C, zero, 0)
        plsc.subcore_barrier()

        pltpu.sync_copy(dst_hbm.at[wid], dst_i)

        def body(b, _):
            pltpu.sync_copy(ones, acc.at[dst_i.at[b]], add=True)
            return 0
        lax.fori_loop(0, CPT, body, 0)
        plsc.subcore_barrier()

        pltpu.sync_copy(acc.at[pl.ds(sid * RPS, RPS)],
                        out_hbm.at[cid, pl.ds(sid * RPS, RPS)])

    return k(dstp)


def _sc_scatter(g, srcp, dstp, d):
    @functools.partial(
        pl.kernel,
        out_type=jax.ShapeDtypeStruct((NC, NP, d), jnp.float32),
        mesh=_sc_mesh(),
        compiler_params=pltpu.CompilerParams(use_tc_tiling_on_sc=False),
        scratch_types=[
            pltpu.VMEM_SHARED((NP, d), jnp.float32),
            pltpu.VMEM((KB, C), jnp.int32),
            pltpu.VMEM((KB, C), jnp.int32),
            pltpu.VMEM((C, d), jnp.float32),
            pltpu.VMEM((C, d), jnp.float32),
            pltpu.VMEM((C, d), jnp.float32),
            pltpu.VMEM((C, d), jnp.float32),
            pltpu.SemaphoreType.DMA,
            pltpu.SemaphoreType.DMA,
            pltpu.SemaphoreType.DMA,
            pltpu.SemaphoreType.DMA,
            pltpu.SemaphoreType.DMA,
            pltpu.SemaphoreType.DMA,
            pltpu.SemaphoreType.DMA,
            pltpu.SemaphoreType.DMA,
        ],
    )
    def k(g_hbm, src_hbm, dst_hbm, out_hbm,
          acc, src_i, dst_i, rows0, rows1, rows2, rows3,
          gs0, gs1, gs2, gs3, ss0, ss1, ss2, ss3):
        cid = lax.axis_index("c")
        sid = lax.axis_index("s")
        wid = cid * NS + sid
        rows = (rows0, rows1, rows2, rows3)
        gs = (gs0, gs1, gs2, gs3)
        ss = (ss0, ss1, ss2, ss3)

        def fill(i, _):
            rows0[i // (d // L), pl.ds((i % (d // L)) * L, L)] = (
                jnp.zeros((L,), jnp.float32))
            return 0
        lax.fori_loop(0, C * (d // L), fill, 0)

        def zero(i, _):
            pltpu.sync_copy(rows0, acc.at[pl.ds(sid * RPS + i * C, C)])
            return 0
        lax.fori_loop(0, RPS // C, zero, 0)
        plsc.subcore_barrier()

        def gat(b, j):
            pltpu.async_copy(g_hbm.at[src_i.at[b]], rows[j], gs[j])

        def gat_wait(j):
            pltpu.make_async_copy(g_hbm.at[pl.ds(0, C)], rows[j], gs[j]).wait()

        def scat(b, j):
            pltpu.async_copy(rows[j], acc.at[dst_i.at[b]], ss[j], add=True)

        def scat_wait(j):
            pltpu.make_async_copy(rows[j], acc.at[dst_i.at[0]], ss[j]).wait()

        def block(blk, _):
            pltpu.sync_copy(src_hbm.at[wid, pl.ds(blk * KB, KB)], src_i)
            pltpu.sync_copy(dst_hbm.at[wid, pl.ds(blk * KB, KB)], dst_i)
            for t in range(3):
                gat(t, t)

            def quad(q, _):
                for j in range(4):
                    b = 4 * q + j
                    if j == 0:
                        @pl.when(q > 0)
                        def _():
                            scat_wait(3)
                        gat(b + 3, 3)
                    else:
                        scat_wait(j - 1)

                        @pl.when(b + 3 < KB)
                        def _(b=b, j=j):
                            gat(b + 3, j - 1)
                    gat_wait(j)
                    scat(b, j)
                return 0
            lax.fori_loop(0, KB // 4, quad, 0)
            scat_wait(3)
            return 0
        lax.fori_loop(0, CPT // KB, block, 0)
        plsc.subcore_barrier()

        pltpu.sync_copy(acc.at[pl.ds(sid * RPS, RPS)],
                        out_hbm.at[cid, pl.ds(sid * RPS, RPS)])

    return k(g, srcp, dstp)


def _tc_prep(x, W1, deg0, deg1):
    def body(x_ref, w_ref, d0_ref, d1_ref, g_ref, dis_ref):
        deg = d0_ref[...] + d1_ref[...] + 1.0
        dis = lax.rsqrt(deg)
        g_ref[...] = jnp.dot(x_ref[...], w_ref[...],
                             preferred_element_type=jnp.float32) * dis
        dis_ref[...] = dis

    return pl.pallas_call(
        body,
        grid=(N // BT,),
        in_specs=[
            pl.BlockSpec((BT, 47), lambda i: (i, 0)),
            pl.BlockSpec((47, 16), lambda i: (0, 0)),
            pl.BlockSpec((BT, 1), lambda i: (i, 0)),
            pl.BlockSpec((BT, 1), lambda i: (i, 0)),
        ],
        out_specs=[
            pl.BlockSpec((BT, 16), lambda i: (i, 0)),
            pl.BlockSpec((BT, 1), lambda i: (i, 0)),
        ],
        out_shape=[
            jax.ShapeDtypeStruct((N, 16), jnp.float32),
            jax.ShapeDtypeStruct((N, 1), jnp.float32),
        ],
    )(x, W1, deg0, deg1)


def _tc_mid(s1p, g1, dis, b1, W2):
    def body(s_ref, g_ref, dis_ref, b_ref, w_ref, out_ref):
        s = s_ref[0] + s_ref[1] + g_ref[...]
        h = jnp.maximum(dis_ref[...] * s + b_ref[...], 0.0)
        out_ref[...] = jnp.dot(h, w_ref[...],
                               preferred_element_type=jnp.float32) * dis_ref[...]

    return pl.pallas_call(
        body,
        grid=(N // BT,),
        in_specs=[
            pl.BlockSpec((2, BT, 16), lambda i: (0, i, 0)),
            pl.BlockSpec((BT, 16), lambda i: (i, 0)),
            pl.BlockSpec((BT, 1), lambda i: (i, 0)),
            pl.BlockSpec((1, 16), lambda i: (0, 0)),
            pl.BlockSpec((16, 32), lambda i: (0, 0)),
        ],
        out_specs=pl.BlockSpec((BT, 32), lambda i: (i, 0)),
        out_shape=jax.ShapeDtypeStruct((N, 32), jnp.float32),
    )(s1p, g1, dis, b1.reshape(1, 16), W2)


def _tc_final(s2p, g2, dis, b2, batch_col,
              Wr1, br1, Wr2, br2, Wr3, br3, Wr4, br4):
    nb = N // BT

    def body(s_ref, g_ref, dis_ref, b_ref, bat_ref,
             w1, c1, w2, c2, w3, c3, w4, c4, out_ref, acc, cnt):
        i = pl.program_id(0)
        s = s_ref[0] + s_ref[1] + g_ref[...]
        h = jnp.maximum(dis_ref[...] * s + b_ref[...], 0.0)
        ids = lax.broadcasted_iota(jnp.int32, (1, NG), 1)
        a = (bat_ref[...] == ids).astype(jnp.float32)
        dn = (((0,), (0,)), ((), ()))
        contrib = lax.dot_general(a, h, dn,
                                  preferred_element_type=jnp.float32)
        ccontrib = lax.dot_general(a, jnp.ones((BT, 1), jnp.float32), dn,
                                   preferred_element_type=jnp.float32)

        @pl.when(i == 0)
        def _():
            acc[...] = contrib
            cnt[...] = ccontrib

        @pl.when(i > 0)
        def _():
            acc[...] += contrib
            cnt[...] += ccontrib

        @pl.when(i == nb - 1)
        def _():
            pooled = acc[...] / jnp.maximum(cnt[...], 1.0)
            r = jnp.maximum(jnp.dot(pooled, w1[...],
                                    preferred_element_type=jnp.float32) + c1[...], 0.0)
            r = jnp.maximum(jnp.dot(r, w2[...],
                                    preferred_element_type=jnp.float32) + c2[...], 0.0)
            r = jnp.maximum(jnp.dot(r, w3[...],
                                    preferred_element_type=jnp.float32) + c3[...], 0.0)
            out_ref[...] = jnp.dot(r, w4[...],
                                   preferred_element_type=jnp.float32) + c4[...]

    return pl.pallas_call(
        body,
        grid=(nb,),
        in_specs=[
            pl.BlockSpec((2, BT, 32), lambda i: (0, i, 0)),
            pl.BlockSpec((BT, 32), lambda i: (i, 0)),
            pl.BlockSpec((BT, 1), lambda i: (i, 0)),
            pl.BlockSpec((1, 32), lambda i: (0, 0)),
            pl.BlockSpec((BT, 1), lambda i: (i, 0)),
            pl.BlockSpec((32, 32), lambda i: (0, 0)),
            pl.BlockSpec((1, 32), lambda i: (0, 0)),
            pl.BlockSpec((32, 16), lambda i: (0, 0)),
            pl.BlockSpec((1, 16), lambda i: (0, 0)),
            pl.BlockSpec((16, 8), lambda i: (0, 0)),
            pl.BlockSpec((1, 8), lambda i: (0, 0)),
            pl.BlockSpec((8, 1), lambda i: (0, 0)),
            pl.BlockSpec((1, 1), lambda i: (0, 0)),
        ],
        out_specs=pl.BlockSpec((NG, 1), lambda i: (0, 0)),
        out_shape=jax.ShapeDtypeStruct((NG, 1), jnp.float32),
        scratch_shapes=[
            pltpu.VMEM((NG, 32), jnp.float32),
            pltpu.VMEM((NG, 1), jnp.float32),
        ],
    )(s2p, g2, dis, b2.reshape(1, 32), batch_col,
      Wr1, br1.reshape(1, 32), Wr2, br2.reshape(1, 16),
      Wr3, br3.reshape(1, 8), Wr4, br4.reshape(1, 1))


def kernel(x, edge_index, batch, W1, b1, W2, b2,
           Wr1, br1, Wr2, br2, Wr3, br3, Wr4, br4):
    src = edge_index[0].astype(jnp.int32)
    dst = edge_index[1].astype(jnp.int32)
    batch_col = batch.astype(jnp.int32).reshape(N, 1)

    npad = EPAD - E
    src_p = jnp.concatenate(
        [src, jnp.zeros((npad,), jnp.int32)]).reshape(NW, CPT, C)
    dst_p = jnp.concatenate(
        [dst, N + (jnp.arange(npad, dtype=jnp.int32) % (NP - N))]
    ).reshape(NW, CPT, C)

    degp = _sc_degree(dst_p)
    deg0 = degp[0].reshape(NP, 1)
    deg1 = degp[1].reshape(NP, 1)
    g1, dis = _tc_prep(x, W1, deg0, deg1)
    s1p = _sc_scatter(g1, src_p, dst_p, 16)
    g2 = _tc_mid(s1p, g1, dis, b1, W2)
    s2p = _sc_scatter(g2, src_p, dst_p, 32)
    return _tc_final(s2p, g2, dis, b2, batch_col,
                     Wr1, br1, Wr2, br2, Wr3, br3, Wr4, br4)


# speedup vs baseline: 66.2350x; 1.1468x over previous
import functools

import jax
import jax.numpy as jnp
from jax import lax
from jax.experimental import pallas as pl
from jax.experimental.pallas import tpu as pltpu
from jax.experimental.pallas import tpu_sc as plsc

NC = 2
NS = 16
NW = NC * NS
L = 16

N = 50000
E = 1_600_000
NG = 512

C = 128
CPT = 392
KB = 28
EPAD = NW * CPT * C
NP = 51200
RPS = NP // NS

BT = 2048


def _sc_mesh():
    return plsc.VectorSubcoreMesh(
        core_axis_name="c", subcore_axis_name="s",
        num_cores=NC, num_subcores=NS)


def _sc_degree(dstp):
    @functools.partial(
        pl.kernel,
        out_type=jax.ShapeDtypeStruct((NC, NP), jnp.float32),
        mesh=_sc_mesh(),
        compiler_params=pltpu.CompilerParams(use_tc_tiling_on_sc=False),
        scratch_types=[
            pltpu.VMEM_SHARED((NP,), jnp.float32),
            pltpu.VMEM((CPT, C), jnp.int32),
            pltpu.VMEM((C,), jnp.float32),
            pltpu.VMEM((C,), jnp.float32),
            pltpu.SemaphoreType.DMA,
        ],
    )
    def k(dst_hbm, out_hbm, acc, dst_i, ones, zeros, dsem):
        cid = lax.axis_index("c")
        sid = lax.axis_index("s")
        wid = cid * NS + sid

        def fill(i, _):
            ones[pl.ds(i * L, L)] = jnp.full((L,), 1.0, jnp.float32)
            zeros[pl.ds(i * L, L)] = jnp.zeros((L,), jnp.float32)
            return 0
        lax.fori_loop(0, C // L, fill, 0)

        def zero(i, _):
            pltpu.sync_copy(zeros, acc.at[pl.ds(sid * RPS + i * C, C)])
            return 0
        lax.fori_loop(0, RPS // C, zero, 0)
        plsc.subcore_barrier()

        pltpu.sync_copy(dst_hbm.at[wid], dst_i)

        def body(b, _):
            pltpu.sync_copy(ones, acc.at[dst_i.at[b]], add=True)
            return 0
        lax.fori_loop(0, CPT, body, 0)
        plsc.subcore_barrier()

        pltpu.sync_copy(acc.at[pl.ds(sid * RPS, RPS)],
                        out_hbm.at[cid, pl.ds(sid * RPS, RPS)])

    return k(dstp)


def _sc_scatter(g, srcp, dstp, d):
    @functools.partial(
        pl.kernel,
        out_type=jax.ShapeDtypeStruct((NC, NP, d), jnp.float32),
        mesh=_sc_mesh(),
        compiler_params=pltpu.CompilerParams(use_tc_tiling_on_sc=False),
        scratch_types=[
            pltpu.VMEM_SHARED((NP, d), jnp.float32),
            pltpu.VMEM((KB, C), jnp.int32),
            pltpu.VMEM((KB, C), jnp.int32),
            pltpu.VMEM((C, d), jnp.float32),
            pltpu.VMEM((C, d), jnp.float32),
            pltpu.VMEM((C, d), jnp.float32),
            pltpu.VMEM((C, d), jnp.float32),
            pltpu.SemaphoreType.DMA,
            pltpu.SemaphoreType.DMA,
            pltpu.SemaphoreType.DMA,
            pltpu.SemaphoreType.DMA,
            pltpu.SemaphoreType.DMA,
            pltpu.SemaphoreType.DMA,
            pltpu.SemaphoreType.DMA,
            pltpu.SemaphoreType.DMA,
        ],
    )
    def k(g_hbm, src_hbm, dst_hbm, out_hbm,
          acc, src_i, dst_i, rows0, rows1, rows2, rows3,
          gs0, gs1, gs2, gs3, ss0, ss1, ss2, ss3):
        cid = lax.axis_index("c")
        sid = lax.axis_index("s")
        wid = cid * NS + sid
        rows = (rows0, rows1, rows2, rows3)
        gs = (gs0, gs1, gs2, gs3)
        ss = (ss0, ss1, ss2, ss3)

        def fill(i, _):
            rows0[i // (d // L), pl.ds((i % (d // L)) * L, L)] = (
                jnp.zeros((L,), jnp.float32))
            return 0
        lax.fori_loop(0, C * (d // L), fill, 0)

        def zero(i, _):
            pltpu.sync_copy(rows0, acc.at[pl.ds(sid * RPS + i * C, C)])
            return 0
        lax.fori_loop(0, RPS // C, zero, 0)
        plsc.subcore_barrier()

        def gat(b, j):
            pltpu.async_copy(g_hbm.at[src_i.at[b]], rows[j], gs[j])

        def gat_wait(j):
            pltpu.make_async_copy(g_hbm.at[pl.ds(0, C)], rows[j], gs[j]).wait()

        def scat(b, j):
            pltpu.async_copy(rows[j], acc.at[dst_i.at[b]], ss[j], add=True)

        def scat_wait(j):
            pltpu.make_async_copy(rows[j], acc.at[dst_i.at[0]], ss[j]).wait()

        def block(blk, _):
            pltpu.sync_copy(src_hbm.at[wid, pl.ds(blk * KB, KB)], src_i)
            pltpu.sync_copy(dst_hbm.at[wid, pl.ds(blk * KB, KB)], dst_i)
            for t in range(3):
                gat(t, t)

            def quad(q, _):
                for j in range(4):
                    b = 4 * q + j
                    if j == 0:
                        @pl.when(q > 0)
                        def _():
                            scat_wait(3)
                        gat(b + 3, 3)
                    else:
                        scat_wait(j - 1)

                        @pl.when(b + 3 < KB)
                        def _(b=b, j=j):
                            gat(b + 3, j - 1)
                    gat_wait(j)
                    scat(b, j)
                return 0
            lax.fori_loop(0, KB // 4, quad, 0)
            scat_wait(3)
            return 0
        lax.fori_loop(0, CPT // KB, block, 0)
        plsc.subcore_barrier()

        pltpu.sync_copy(acc.at[pl.ds(sid * RPS, RPS)],
                        out_hbm.at[cid, pl.ds(sid * RPS, RPS)])

    return k(g, srcp, dstp)


def _dis_of(d0, d1):
    return lax.rsqrt(d0 + d1 + 1.0)


def _tc_prep(x_pad, W1, degp):
    def body(x_ref, w_ref, deg_ref, g_ref):
        dis = _dis_of(deg_ref[0], deg_ref[1])[:, None]
        g_ref[...] = jnp.dot(x_ref[...], w_ref[...],
                             preferred_element_type=jnp.float32) * dis

    return pl.pallas_call(
        body,
        grid=(NP // BT,),
        in_specs=[
            pl.BlockSpec((BT, 47), lambda i: (i, 0)),
            pl.BlockSpec((47, 16), lambda i: (0, 0)),
            pl.BlockSpec((2, BT), lambda i: (0, i)),
        ],
        out_specs=pl.BlockSpec((BT, 16), lambda i: (i, 0)),
        out_shape=jax.ShapeDtypeStruct((NP, 16), jnp.float32),
    )(x_pad, W1, degp)


def _tc_mid(s1p, g1, degp, b1, W2):
    def body(s_ref, g_ref, deg_ref, b_ref, w_ref, out_ref):
        dis = _dis_of(deg_ref[0], deg_ref[1])[:, None]
        s = s_ref[0] + s_ref[1] + g_ref[...]
        h = jnp.maximum(dis * s + b_ref[...], 0.0)
        out_ref[...] = jnp.dot(h, w_ref[...],
                               preferred_element_type=jnp.float32) * dis

    return pl.pallas_call(
        body,
        grid=(NP // BT,),
        in_specs=[
            pl.BlockSpec((2, BT, 16), lambda i: (0, i, 0)),
            pl.BlockSpec((BT, 16), lambda i: (i, 0)),
            pl.BlockSpec((2, BT), lambda i: (0, i)),
            pl.BlockSpec((1, 16), lambda i: (0, 0)),
            pl.BlockSpec((16, 32), lambda i: (0, 0)),
        ],
        out_specs=pl.BlockSpec((BT, 32), lambda i: (i, 0)),
        out_shape=jax.ShapeDtypeStruct((NP, 32), jnp.float32),
    )(s1p, g1, degp, b1.reshape(1, 16), W2)


def _tc_final(s2p, g2, degp, b2, batch_col,
              Wr1, br1, Wr2, br2, Wr3, br3, Wr4, br4):
    nb = NP // BT

    def body(s_ref, g_ref, deg_ref, b_ref, bat_ref,
             w1, c1, w2, c2, w3, c3, w4, c4, out_ref, acc, cnt):
        i = pl.program_id(0)
        dis = _dis_of(deg_ref[0], deg_ref[1])[:, None]
        s = s_ref[0] + s_ref[1] + g_ref[...]
        h = jnp.maximum(dis * s + b_ref[...], 0.0)
        ids = lax.broadcasted_iota(jnp.int32, (1, NG), 1)
        a = (bat_ref[...] == ids).astype(jnp.float32)
        dn = (((0,), (0,)), ((), ()))
        contrib = lax.dot_general(a, h, dn,
                                  preferred_element_type=jnp.float32)
        ccontrib = lax.dot_general(a, jnp.ones((BT, 1), jnp.float32), dn,
                                   preferred_element_type=jnp.float32)

        @pl.when(i == 0)
        def _():
            acc[...] = contrib
            cnt[...] = ccontrib

        @pl.when(i > 0)
        def _():
            acc[...] += contrib
            cnt[...] += ccontrib

        @pl.when(i == nb - 1)
        def _():
            pooled = acc[...] / jnp.maximum(cnt[...], 1.0)
            r = jnp.maximum(jnp.dot(pooled, w1[...],
                                    preferred_element_type=jnp.float32) + c1[...], 0.0)
            r = jnp.maximum(jnp.dot(r, w2[...],
                                    preferred_element_type=jnp.float32) + c2[...], 0.0)
            r = jnp.maximum(jnp.dot(r, w3[...],
                                    preferred_element_type=jnp.float32) + c3[...], 0.0)
            out_ref[...] = jnp.dot(r, w4[...],
                                   preferred_element_type=jnp.float32) + c4[...]

    return pl.pallas_call(
        body,
        grid=(nb,),
        in_specs=[
            pl.BlockSpec((2, BT, 32), lambda i: (0, i, 0)),
            pl.BlockSpec((BT, 32), lambda i: (i, 0)),
            pl.BlockSpec((2, BT), lambda i: (0, i)),
            pl.BlockSpec((1, 32), lambda i: (0, 0)),
            pl.BlockSpec((BT, 1), lambda i: (i, 0)),
            pl.BlockSpec((32, 32), lambda i: (0, 0)),
            pl.BlockSpec((1, 32), lambda i: (0, 0)),
            pl.BlockSpec((32, 16), lambda i: (0, 0)),
            pl.BlockSpec((1, 16), lambda i: (0, 0)),
            pl.BlockSpec((16, 8), lambda i: (0, 0)),
            pl.BlockSpec((1, 8), lambda i: (0, 0)),
            pl.BlockSpec((8, 1), lambda i: (0, 0)),
            pl.BlockSpec((1, 1), lambda i: (0, 0)),
        ],
        out_specs=pl.BlockSpec((NG, 1), lambda i: (0, 0)),
        out_shape=jax.ShapeDtypeStruct((NG, 1), jnp.float32),
        scratch_shapes=[
            pltpu.VMEM((NG, 32), jnp.float32),
            pltpu.VMEM((NG, 1), jnp.float32),
        ],
    )(s2p, g2, degp, b2.reshape(1, 32), batch_col,
      Wr1, br1.reshape(1, 32), Wr2, br2.reshape(1, 16),
      Wr3, br3.reshape(1, 8), Wr4, br4.reshape(1, 1))


def kernel(x, edge_index, batch, W1, b1, W2, b2,
           Wr1, br1, Wr2, br2, Wr3, br3, Wr4, br4):
    src = edge_index[0].astype(jnp.int32)
    dst = edge_index[1].astype(jnp.int32)
    batch_col = jnp.concatenate(
        [batch.astype(jnp.int32),
         jnp.full((NP - N,), NG, jnp.int32)]).reshape(NP, 1)
    x_pad = jnp.pad(x, ((0, NP - N), (0, 0)))

    npad = EPAD - E
    src_p = jnp.concatenate(
        [src, jnp.zeros((npad,), jnp.int32)]).reshape(NW, CPT, C)
    dst_p = jnp.concatenate(
        [dst, N + (jnp.arange(npad, dtype=jnp.int32) % (NP - N))]
    ).reshape(NW, CPT, C)

    degp = _sc_degree(dst_p)
    g1 = _tc_prep(x_pad, W1, degp)
    s1p = _sc_scatter(g1, src_p, dst_p, 16)
    g2 = _tc_mid(s1p, g1, degp, b1, W2)
    s2p = _sc_scatter(g2, src_p, dst_p, 32)
    return _tc_final(s2p, g2, degp, b2, batch_col,
                     Wr1, br1, Wr2, br2, Wr3, br3, Wr4, br4)
